# Initial kernel scaffold; baseline (speedup 1.0000x reference)
import jax
import jax.numpy as jnp
from jax.experimental import pallas as pl


def kernel(x, edge_index, W, b):
    raise NotImplementedError("write your pallas kernel here")


# speedup vs baseline: 10.2559x; 10.2559x over previous
import functools

import jax
import jax.numpy as jnp
from jax import lax
from jax.experimental import pallas as pl
from jax.experimental.pallas import tpu as pltpu
from jax.experimental.pallas import tpu_sc as plsc

NC = 2
NS = 16
NW = NC * NS
L = 16

N_NODES = 10000
D = 128
DH = D // NC
NP = 10240
N_EDGES = 320000
CH = 128
NCH = 160
EPT = CH * NCH
EP = NS * EPT
EPW = EP // NW
RPS = NP // NS

_mesh = plsc.VectorSubcoreMesh(core_axis_name="c", subcore_axis_name="s")


@functools.partial(
    pl.kernel,
    out_type=jax.ShapeDtypeStruct((2, NW, NP), jnp.float32),
    mesh=_mesh,
    scratch_types=[
        pltpu.VMEM((EPW,), jnp.int32),
        pltpu.VMEM((NP,), jnp.float32),
        pltpu.VMEM((NP,), jnp.float32),
    ],
    compiler_params=pltpu.CompilerParams(needs_layout_passes=False),
)
def _sc_degrees(src_hbm, dst_hbm, deg_hbm, idx_v, hs_v, hd_v):
    cid = lax.axis_index("c")
    sid = lax.axis_index("s")
    wid = sid * NC + cid
    zero = jnp.zeros((L,), jnp.float32)
    one = jnp.ones((L,), jnp.float32)

    @pl.loop(0, NP // L)
    def _zero(i):
        o = pl.multiple_of(i * L, L)
        hs_v[pl.ds(o, L)] = zero
        hd_v[pl.ds(o, L)] = zero

    base = wid * EPW
    pltpu.sync_copy(src_hbm.at[pl.ds(base, EPW)], idx_v)

    @pl.loop(0, EPW // L)
    def _hist_src(i):
        o = pl.multiple_of(i * L, L)
        plsc.addupdate_scatter(hs_v, [idx_v[pl.ds(o, L)]], one)

    pltpu.sync_copy(dst_hbm.at[pl.ds(base, EPW)], idx_v)

    @pl.loop(0, EPW // L)
    def _hist_dst(i):
        o = pl.multiple_of(i * L, L)
        plsc.addupdate_scatter(hd_v, [idx_v[pl.ds(o, L)]], one)

    pltpu.sync_copy(hs_v, deg_hbm.at[0, wid])
    pltpu.sync_copy(hd_v, deg_hbm.at[1, wid])


@functools.partial(
    pl.kernel,
    out_type=jax.ShapeDtypeStruct((NC, NP, DH), jnp.float32),
    mesh=_mesh,
    scratch_types=[
        pltpu.VMEM((NCH, CH), jnp.int32),
        pltpu.VMEM((NCH, CH), jnp.int32),
        pltpu.VMEM((CH, DH), jnp.float32),
        pltpu.VMEM((CH, DH), jnp.float32),
        pltpu.VMEM_SHARED((NP, DH), jnp.float32),
        pltpu.SemaphoreType.DMA,
        pltpu.SemaphoreType.DMA,
    ],
    compiler_params=pltpu.CompilerParams(use_tc_tiling_on_sc=False),
)
def _sc_scatter(xs_lo_hbm, xs_hi_hbm, src_hbm, dst_hbm, zeros_hbm, out_hbm,
                src_v, dst_v, rows0, rows1, acc, sem0, sem1):
    cid = lax.axis_index("c")
    sid = lax.axis_index("s")
    rb = sid * RPS

    pltpu.sync_copy(zeros_hbm, acc.at[pl.ds(rb, RPS)])
    pltpu.sync_copy(src_hbm.at[sid], src_v)
    pltpu.sync_copy(dst_hbm.at[sid], dst_v)
    plsc.subcore_barrier()

    def _edge_loop(xs_hbm):
        pltpu.async_copy(xs_hbm.at[src_v.at[0]], rows0, sem0)

        @pl.loop(0, NCH // 2)
        def _chunks(i):
            c0 = i * 2
            pltpu.make_async_copy(xs_hbm.at[src_v.at[c0]], rows0, sem0).wait()
            pltpu.async_copy(xs_hbm.at[src_v.at[c0 + 1]], rows1, sem1)
            pltpu.sync_copy(rows0, acc.at[dst_v.at[c0]], add=True)
            pltpu.make_async_copy(
                xs_hbm.at[src_v.at[c0 + 1]], rows1, sem1).wait()

            @pl.when(c0 + 2 < NCH)
            def _():
                pltpu.async_copy(xs_hbm.at[src_v.at[c0 + 2]], rows0, sem0)

            pltpu.sync_copy(rows1, acc.at[dst_v.at[c0 + 1]], add=True)

    @pl.when(cid == 0)
    def _lo():
        _edge_loop(xs_lo_hbm)

    @pl.when(cid == 1)
    def _hi():
        _edge_loop(xs_hi_hbm)

    plsc.subcore_barrier()
    pltpu.sync_copy(acc.at[pl.ds(rb, RPS)], out_hbm.at[cid, pl.ds(rb, RPS)])


def _scale_body(deg_ref, x_ref, o_ref):
    d = jnp.sum(deg_ref[0], axis=0)
    nrm = lax.rsqrt(jnp.maximum(d, 1.0))
    xs = x_ref[...] * nrm[:, None]
    o_ref[0] = xs[:, :DH]
    o_ref[1] = xs[:, DH:]


_BLK = 1024
_tc_scale = pl.pallas_call(
    _scale_body,
    grid=(NP // _BLK,),
    in_specs=[
        pl.BlockSpec((1, NW, _BLK), lambda i: (0, 0, i)),
        pl.BlockSpec((_BLK, D), lambda i: (i, 0)),
    ],
    out_specs=pl.BlockSpec((NC, _BLK, DH), lambda i: (0, i, 0)),
    out_shape=jax.ShapeDtypeStruct((NC, NP, DH), jnp.float32),
)


def _final_body(p_ref, deg_ref, w_ref, b_ref, o_ref):
    s = jnp.concatenate([p_ref[0], p_ref[1]], axis=-1)
    d = jnp.sum(deg_ref[0], axis=0)
    nrm = lax.rsqrt(jnp.maximum(d, 1.0))
    h = jnp.dot(s, w_ref[...], preferred_element_type=jnp.float32,
                precision=lax.Precision.HIGHEST)
    o_ref[...] = h * nrm[:, None] + b_ref[0][None, :]


_BLKO = 1024
_tc_final = pl.pallas_call(
    _final_body,
    grid=(pl.cdiv(N_NODES, _BLKO),),
    in_specs=[
        pl.BlockSpec((NC, _BLKO, DH), lambda i: (0, i, 0)),
        pl.BlockSpec((1, NW, _BLKO), lambda i: (1, 0, i)),
        pl.BlockSpec((D, D), lambda i: (0, 0)),
        pl.BlockSpec((1, D), lambda i: (0, 0)),
    ],
    out_specs=pl.BlockSpec((_BLKO, D), lambda i: (i, 0)),
    out_shape=jax.ShapeDtypeStruct((N_NODES, D), jnp.float32),
)


def kernel(x, edge_index, W, b):
    src = edge_index[0]
    dst = edge_index[1]
    pad = jnp.full((EP - N_EDGES,), N_NODES, dtype=jnp.int32)
    src_p = jnp.concatenate([src, pad])
    dst_p = jnp.concatenate([dst, pad])

    deg = _sc_degrees(src_p, dst_p)

    x_p = jnp.concatenate([x, jnp.zeros((NP - N_NODES, D), x.dtype)])
    xs = _tc_scale(deg, x_p)

    zeros2 = jnp.zeros((RPS, DH), jnp.float32)
    parts = _sc_scatter(xs[0], xs[1], src_p.reshape(NS, NCH, CH),
                        dst_p.reshape(NS, NCH, CH), zeros2)

    return _tc_final(parts, deg, W, b.reshape(1, D))


# speedup vs baseline: 11.2672x; 1.0986x over previous
import functools

import jax
import jax.numpy as jnp
from jax import lax
from jax.experimental import pallas as pl
from jax.experimental.pallas import tpu as pltpu
from jax.experimental.pallas import tpu_sc as plsc

NC = 2
NS = 16
NW = NC * NS
L = 16

N_NODES = 10000
D = 128
DH = D // NC
NP = 10240
N_EDGES = 320000
CH = 128
NCH = 160
EPT = CH * NCH
EP = NS * EPT
EPW = EP // NW
RPS = NP // NS

_mesh = plsc.VectorSubcoreMesh(core_axis_name="c", subcore_axis_name="s")


@functools.partial(
    pl.kernel,
    out_type=jax.ShapeDtypeStruct((2, NW, NP), jnp.float32),
    mesh=_mesh,
    scratch_types=[
        pltpu.VMEM((EPW,), jnp.int32),
        pltpu.VMEM((EPW,), jnp.int32),
        pltpu.VMEM((NP,), jnp.float32),
        pltpu.VMEM((NP,), jnp.float32),
        pltpu.SemaphoreType.DMA,
    ],
    compiler_params=pltpu.CompilerParams(needs_layout_passes=False),
)
def _sc_degrees(src_hbm, dst_hbm, zeros_hbm, deg_hbm,
                si_v, di_v, hs_v, hd_v, sem):
    cid = lax.axis_index("c")
    sid = lax.axis_index("s")
    wid = sid * NC + cid
    one = jnp.ones((L,), jnp.float32)

    base = wid * EPW
    cs = pltpu.async_copy(src_hbm.at[pl.ds(base, EPW)], si_v, sem)
    cd = pltpu.async_copy(dst_hbm.at[pl.ds(base, EPW)], di_v, sem)
    cz0 = pltpu.async_copy(zeros_hbm, hs_v, sem)
    cz1 = pltpu.async_copy(zeros_hbm, hd_v, sem)
    cs.wait()
    cd.wait()
    cz0.wait()
    cz1.wait()

    @pl.loop(0, EPW // L, unroll=8)
    def _hist(i):
        o = pl.multiple_of(i * L, L)
        plsc.addupdate_scatter(hs_v, [si_v[pl.ds(o, L)]], one)
        plsc.addupdate_scatter(hd_v, [di_v[pl.ds(o, L)]], one)

    pltpu.sync_copy(hs_v, deg_hbm.at[0, wid])
    pltpu.sync_copy(hd_v, deg_hbm.at[1, wid])


@functools.partial(
    pl.kernel,
    out_type=jax.ShapeDtypeStruct((NC, NP, DH), jnp.float32),
    mesh=_mesh,
    scratch_types=[
        pltpu.VMEM((NCH, CH), jnp.int32),
        pltpu.VMEM((NCH, CH), jnp.int32),
        pltpu.VMEM((CH, DH), jnp.float32),
        pltpu.VMEM((CH, DH), jnp.float32),
        pltpu.VMEM((CH, DH), jnp.float32),
        pltpu.VMEM((CH, DH), jnp.float32),
        pltpu.VMEM_SHARED((NP, DH), jnp.float32),
        pltpu.SemaphoreType.DMA,
        pltpu.SemaphoreType.DMA,
        pltpu.SemaphoreType.DMA,
        pltpu.SemaphoreType.DMA,
    ],
    compiler_params=pltpu.CompilerParams(use_tc_tiling_on_sc=False),
)
def _sc_scatter(xs_lo_hbm, xs_hi_hbm, src_hbm, dst_hbm, zeros_hbm, out_hbm,
                src_v, dst_v, rows0, rows1, rows2, rows3, acc,
                sem0, sem1, sem2, sem3):
    cid = lax.axis_index("c")
    sid = lax.axis_index("s")
    rb = sid * RPS
    rows = (rows0, rows1, rows2, rows3)
    sems = (sem0, sem1, sem2, sem3)
    NB = 4

    pltpu.sync_copy(zeros_hbm, acc.at[pl.ds(rb, RPS)])
    pltpu.sync_copy(src_hbm.at[sid], src_v)
    pltpu.sync_copy(dst_hbm.at[sid], dst_v)
    plsc.subcore_barrier()

    def _edge_loop(xs_hbm):
        for j in range(NB):
            pltpu.async_copy(xs_hbm.at[src_v.at[j]], rows[j], sems[j])

        @pl.loop(0, NCH // NB)
        def _chunks(i):
            c0 = i * NB
            for j in range(NB):
                pltpu.make_async_copy(
                    xs_hbm.at[src_v.at[c0 + j]], rows[j], sems[j]).wait()
                pltpu.async_copy(
                    rows[j], acc.at[dst_v.at[c0 + j]], sems[j], add=True)
            for j in range(NB):
                pltpu.make_async_copy(
                    rows[j], acc.at[dst_v.at[c0 + j]], sems[j]).wait()

                @pl.when(c0 + NB + j < NCH)
                def _():
                    pltpu.async_copy(
                        xs_hbm.at[src_v.at[c0 + NB + j]], rows[j], sems[j])

    @pl.when(cid == 0)
    def _lo():
        _edge_loop(xs_lo_hbm)

    @pl.when(cid == 1)
    def _hi():
        _edge_loop(xs_hi_hbm)

    plsc.subcore_barrier()
    pltpu.sync_copy(acc.at[pl.ds(rb, RPS)], out_hbm.at[cid, pl.ds(rb, RPS)])


def _scale_body(deg_ref, x_ref, o_ref):
    d = jnp.sum(deg_ref[0], axis=0)
    nrm = lax.rsqrt(jnp.maximum(d, 1.0))
    xs = x_ref[...] * nrm[:, None]
    o_ref[0] = xs[:, :DH]
    o_ref[1] = xs[:, DH:]


_BLK = 1024
_tc_scale = pl.pallas_call(
    _scale_body,
    grid=(NP // _BLK,),
    in_specs=[
        pl.BlockSpec((1, NW, _BLK), lambda i: (0, 0, i)),
        pl.BlockSpec((_BLK, D), lambda i: (i, 0)),
    ],
    out_specs=pl.BlockSpec((NC, _BLK, DH), lambda i: (0, i, 0)),
    out_shape=jax.ShapeDtypeStruct((NC, NP, DH), jnp.float32),
)


def _final_body(p_ref, deg_ref, w_ref, b_ref, o_ref):
    s = jnp.concatenate([p_ref[0], p_ref[1]], axis=-1)
    d = jnp.sum(deg_ref[0], axis=0)
    nrm = lax.rsqrt(jnp.maximum(d, 1.0))
    h = jnp.dot(s, w_ref[...], preferred_element_type=jnp.float32,
                precision=lax.Precision.HIGHEST)
    o_ref[...] = h * nrm[:, None] + b_ref[0][None, :]


_BLKO = 1024
_tc_final = pl.pallas_call(
    _final_body,
    grid=(pl.cdiv(N_NODES, _BLKO),),
    in_specs=[
        pl.BlockSpec((NC, _BLKO, DH), lambda i: (0, i, 0)),
        pl.BlockSpec((1, NW, _BLKO), lambda i: (1, 0, i)),
        pl.BlockSpec((D, D), lambda i: (0, 0)),
        pl.BlockSpec((1, D), lambda i: (0, 0)),
    ],
    out_specs=pl.BlockSpec((_BLKO, D), lambda i: (i, 0)),
    out_shape=jax.ShapeDtypeStruct((N_NODES, D), jnp.float32),
)


def kernel(x, edge_index, W, b):
    src = edge_index[0]
    dst = edge_index[1]
    pad = jnp.full((EP - N_EDGES,), N_NODES, dtype=jnp.int32)
    src_p = jnp.concatenate([src, pad])
    dst_p = jnp.concatenate([dst, pad])

    deg = _sc_degrees(src_p, dst_p, jnp.zeros((NP,), jnp.float32))

    x_p = jnp.concatenate([x, jnp.zeros((NP - N_NODES, D), x.dtype)])
    xs = _tc_scale(deg, x_p)

    zeros2 = jnp.zeros((RPS, DH), jnp.float32)
    parts = _sc_scatter(xs[0], xs[1], src_p.reshape(NS, NCH, CH),
                        dst_p.reshape(NS, NCH, CH), zeros2)

    return _tc_final(parts, deg, W, b.reshape(1, D))


# speedup vs baseline: 11.3343x; 1.0060x over previous
import functools

import jax
import jax.numpy as jnp
from jax import lax
from jax.experimental import pallas as pl
from jax.experimental.pallas import tpu as pltpu
from jax.experimental.pallas import tpu_sc as plsc

NC = 2
NS = 16
NW = NC * NS
L = 16

N_NODES = 10000
D = 128
DH = D // NC
NP = 10240
N_EDGES = 320000
CH = 128
NCH = 160
EPT = CH * NCH
EP = NS * EPT
EPW = EP // NW
RPS = NP // NS

_mesh = plsc.VectorSubcoreMesh(core_axis_name="c", subcore_axis_name="s")


@functools.partial(
    pl.kernel,
    out_type=jax.ShapeDtypeStruct((2, NW, NP), jnp.float32),
    mesh=_mesh,
    scratch_types=[
        pltpu.VMEM((EPW,), jnp.int32),
        pltpu.VMEM((EPW,), jnp.int32),
        pltpu.VMEM((NP,), jnp.float32),
        pltpu.VMEM((NP,), jnp.float32),
        pltpu.SemaphoreType.DMA,
    ],
    compiler_params=pltpu.CompilerParams(needs_layout_passes=False),
)
def _sc_degrees(src_hbm, dst_hbm, zeros_hbm, deg_hbm,
                si_v, di_v, hs_v, hd_v, sem):
    cid = lax.axis_index("c")
    sid = lax.axis_index("s")
    wid = sid * NC + cid
    one = jnp.ones((L,), jnp.float32)

    base = wid * EPW
    cs = pltpu.async_copy(src_hbm.at[pl.ds(base, EPW)], si_v, sem)
    cd = pltpu.async_copy(dst_hbm.at[pl.ds(base, EPW)], di_v, sem)
    cz0 = pltpu.async_copy(zeros_hbm, hs_v, sem)
    cz1 = pltpu.async_copy(zeros_hbm, hd_v, sem)
    cs.wait()
    cd.wait()
    cz0.wait()
    cz1.wait()

    @pl.loop(0, EPW // L, unroll=8)
    def _hist(i):
        o = pl.multiple_of(i * L, L)
        plsc.addupdate_scatter(hs_v, [si_v[pl.ds(o, L)]], one)
        plsc.addupdate_scatter(hd_v, [di_v[pl.ds(o, L)]], one)

    pltpu.sync_copy(hs_v, deg_hbm.at[0, wid])
    pltpu.sync_copy(hd_v, deg_hbm.at[1, wid])


@functools.partial(
    pl.kernel,
    out_type=jax.ShapeDtypeStruct((NC, NP, DH), jnp.float32),
    mesh=_mesh,
    scratch_types=[
        pltpu.VMEM((NCH, CH), jnp.int32),
        pltpu.VMEM((NCH, CH), jnp.int32),
        pltpu.VMEM((CH, DH), jnp.float32),
        pltpu.VMEM((CH, DH), jnp.float32),
        pltpu.VMEM((CH, DH), jnp.float32),
        pltpu.VMEM((CH, DH), jnp.float32),
        pltpu.VMEM((CH, DH), jnp.float32),
        pltpu.VMEM_SHARED((NP, DH), jnp.float32),
        pltpu.SemaphoreType.DMA,
        pltpu.SemaphoreType.DMA,
        pltpu.SemaphoreType.DMA,
        pltpu.SemaphoreType.DMA,
        pltpu.SemaphoreType.DMA,
    ],
    compiler_params=pltpu.CompilerParams(use_tc_tiling_on_sc=False),
)
def _sc_scatter(xs_lo_hbm, xs_hi_hbm, src_hbm, dst_hbm, zeros_hbm, out_hbm,
                src_v, dst_v, rows0, rows1, rows2, rows3, rows4, acc,
                sem0, sem1, sem2, sem3, sem4):
    cid = lax.axis_index("c")
    sid = lax.axis_index("s")
    rb = sid * RPS
    rows = (rows0, rows1, rows2, rows3, rows4)
    sems = (sem0, sem1, sem2, sem3, sem4)
    NB = 5

    cz = pltpu.async_copy(zeros_hbm, acc.at[pl.ds(rb, RPS)], sem0)
    ci = pltpu.async_copy(src_hbm.at[sid], src_v, sem1)
    cj = pltpu.async_copy(dst_hbm.at[sid], dst_v, sem2)
    cz.wait()
    ci.wait()
    cj.wait()
    plsc.subcore_barrier()

    def _edge_loop(xs_hbm):
        for j in range(NB):
            pltpu.async_copy(xs_hbm.at[src_v.at[j]], rows[j], sems[j])

        @pl.loop(0, NCH // NB)
        def _chunks(i):
            c0 = i * NB
            for j in range(NB):
                pltpu.make_async_copy(
                    xs_hbm.at[src_v.at[c0 + j]], rows[j], sems[j]).wait()
                pltpu.async_copy(
                    rows[j], acc.at[dst_v.at[c0 + j]], sems[j], add=True)
            for j in range(NB):
                pltpu.make_async_copy(
                    rows[j], acc.at[dst_v.at[c0 + j]], sems[j]).wait()

                @pl.when(c0 + NB + j < NCH)
                def _():
                    pltpu.async_copy(
                        xs_hbm.at[src_v.at[c0 + NB + j]], rows[j], sems[j])

    @pl.when(cid == 0)
    def _lo():
        _edge_loop(xs_lo_hbm)

    @pl.when(cid == 1)
    def _hi():
        _edge_loop(xs_hi_hbm)

    plsc.subcore_barrier()
    pltpu.sync_copy(acc.at[pl.ds(rb, RPS)], out_hbm.at[cid, pl.ds(rb, RPS)])


def _scale_body(deg_ref, x_ref, o_ref):
    d = jnp.sum(deg_ref[0], axis=0)
    nrm = lax.rsqrt(jnp.maximum(d, 1.0))
    xs = x_ref[...] * nrm[:, None]
    o_ref[0] = xs[:, :DH]
    o_ref[1] = xs[:, DH:]


_BLK = 1024
_tc_scale = pl.pallas_call(
    _scale_body,
    grid=(NP // _BLK,),
    in_specs=[
        pl.BlockSpec((1, NW, _BLK), lambda i: (0, 0, i)),
        pl.BlockSpec((_BLK, D), lambda i: (i, 0)),
    ],
    out_specs=pl.BlockSpec((NC, _BLK, DH), lambda i: (0, i, 0)),
    out_shape=jax.ShapeDtypeStruct((NC, NP, DH), jnp.float32),
)


def _final_body(p_ref, deg_ref, w_ref, b_ref, o_ref):
    s = jnp.concatenate([p_ref[0], p_ref[1]], axis=-1)
    d = jnp.sum(deg_ref[0], axis=0)
    nrm = lax.rsqrt(jnp.maximum(d, 1.0))
    h = jnp.dot(s, w_ref[...], preferred_element_type=jnp.float32,
                precision=lax.Precision.HIGHEST)
    o_ref[...] = h * nrm[:, None] + b_ref[0][None, :]


_BLKO = 1024
_tc_final = pl.pallas_call(
    _final_body,
    grid=(pl.cdiv(N_NODES, _BLKO),),
    in_specs=[
        pl.BlockSpec((NC, _BLKO, DH), lambda i: (0, i, 0)),
        pl.BlockSpec((1, NW, _BLKO), lambda i: (1, 0, i)),
        pl.BlockSpec((D, D), lambda i: (0, 0)),
        pl.BlockSpec((1, D), lambda i: (0, 0)),
    ],
    out_specs=pl.BlockSpec((_BLKO, D), lambda i: (i, 0)),
    out_shape=jax.ShapeDtypeStruct((N_NODES, D), jnp.float32),
)


def kernel(x, edge_index, W, b):
    src = edge_index[0]
    dst = edge_index[1]
    pad = jnp.full((EP - N_EDGES,), N_NODES, dtype=jnp.int32)
    src_p = jnp.concatenate([src, pad])
    dst_p = jnp.concatenate([dst, pad])

    deg = _sc_degrees(src_p, dst_p, jnp.zeros((NP,), jnp.float32))

    x_p = jnp.concatenate([x, jnp.zeros((NP - N_NODES, D), x.dtype)])
    xs = _tc_scale(deg, x_p)

    zeros2 = jnp.zeros((RPS, DH), jnp.float32)
    parts = _sc_scatter(xs[0], xs[1], src_p.reshape(NS, NCH, CH),
                        dst_p.reshape(NS, NCH, CH), zeros2)

    return _tc_final(parts, deg, W, b.reshape(1, D))


# speedup vs baseline: 11.6555x; 1.0283x over previous
import functools

import jax
import jax.numpy as jnp
from jax import lax
from jax.experimental import pallas as pl
from jax.experimental.pallas import tpu as pltpu
from jax.experimental.pallas import tpu_sc as plsc

NC = 2
NS = 16
L = 16

N_NODES = 10000
D = 128
DH = D // NC
NP = 10240
NPR = NP // L
N_EDGES = 320000
CH = 128
NCH = 160
EPT = CH * NCH
EP = NS * EPT
RPS = NP // NS
HRS = NPR // NS
NB = 4

_mesh = plsc.VectorSubcoreMesh(core_axis_name="c", subcore_axis_name="s")


def _rsqrt16(d):
    i = plsc.bitcast(d, jnp.int32)
    i = 0x5F3759DF - lax.shift_right_logical(i, 1)
    y = plsc.bitcast(i, jnp.float32)
    for _ in range(3):
        y = y * (1.5 - 0.5 * d * y * y)
    return y


@functools.partial(
    pl.kernel,
    out_type=[
        jax.ShapeDtypeStruct((NC, NP, DH), jnp.float32),
        jax.ShapeDtypeStruct((NP, DH), jnp.float32),
        jax.ShapeDtypeStruct((NP, DH), jnp.float32),
        jax.ShapeDtypeStruct((NPR, L), jnp.float32),
    ],
    mesh=_mesh,
    scratch_types=[
        pltpu.VMEM((NCH, CH), jnp.int32),
        pltpu.VMEM((NCH, CH), jnp.int32),
        pltpu.VMEM((CH, DH), jnp.float32),
        pltpu.VMEM((CH, DH), jnp.float32),
        pltpu.VMEM((CH, DH), jnp.float32),
        pltpu.VMEM((CH, DH), jnp.float32),
        pltpu.VMEM((NPR, L), jnp.float32),
        pltpu.VMEM((HRS, L), jnp.float32),
        pltpu.VMEM((HRS, L), jnp.float32),
        pltpu.VMEM((5, CH), jnp.int32),
        pltpu.VMEM_SHARED((NP, DH), jnp.float32),
        pltpu.VMEM_SHARED((NPR, L), jnp.float32),
        pltpu.VMEM_SHARED((NPR, L), jnp.float32),
        pltpu.SemaphoreType.DMA,
        pltpu.SemaphoreType.DMA,
        pltpu.SemaphoreType.DMA,
        pltpu.SemaphoreType.DMA,
    ],
    compiler_params=pltpu.CompilerParams(needs_layout_passes=False,
                                         use_tc_tiling_on_sc=False),
)
def _sc_fused(x_lo_hbm, x_hi_hbm, src_hbm, dst_hbm, z1_hbm, z2_hbm,
              out_hbm, xs_lo_hbm, xs_hi_hbm, nrm_hbm,
              src_v, dst_v, rows0, rows1, rows2, rows3,
              hist_v, deg_v, nsrc_v, idxio,
              acc, hs_s, hd_s, sem0, sem1, sem2, sem3):
    cid = lax.axis_index("c")
    sid = lax.axis_index("s")
    rb = sid * RPS
    hb = sid * HRS
    rows = (rows0, rows1, rows2, rows3)
    sems = (sem0, sem1, sem2, sem3)
    one = jnp.ones((L,), jnp.float32)

    c0 = pltpu.async_copy(z2_hbm, acc.at[pl.ds(rb, RPS)], sem0)
    c1 = pltpu.async_copy(src_hbm.at[sid], src_v, sem1)
    c2 = pltpu.async_copy(dst_hbm.at[sid], dst_v, sem2)
    c3 = pltpu.async_copy(z1_hbm, hist_v, sem3)
    c4 = pltpu.async_copy(z1_hbm.at[pl.ds(hb, HRS)],
                          hs_s.at[pl.ds(hb, HRS)], sem0)
    c5 = pltpu.async_copy(z1_hbm.at[pl.ds(hb, HRS)],
                          hd_s.at[pl.ds(hb, HRS)], sem1)
    for j in range(5):
        for k in range(CH // L):
            idxio[j, pl.ds(k * L, L)] = (
                lax.iota(jnp.int32, L) + (j * CH + k * L))
    c0.wait()
    c1.wait()
    c2.wait()
    c3.wait()
    c4.wait()
    c5.wait()

    @pl.loop(0, EPT // L, unroll=4)
    def _hist_src(i):
        o = pl.multiple_of(i * L, L)
        c = o // CH
        k = o - c * CH
        v = src_v[c, pl.ds(k, L)]
        plsc.addupdate_scatter(
            hist_v, [lax.shift_right_logical(v, 4), v & (L - 1)], one)

    plsc.subcore_barrier()
    for j in range(5):
        pltpu.sync_copy(hist_v.at[pl.ds(j * CH, CH)],
                        hs_s.at[idxio.at[j]], add=True)
    pltpu.sync_copy(z1_hbm, hist_v)

    @pl.loop(0, EPT // L, unroll=4)
    def _hist_dst(i):
        o = pl.multiple_of(i * L, L)
        c = o // CH
        k = o - c * CH
        v = dst_v[c, pl.ds(k, L)]
        plsc.addupdate_scatter(
            hist_v, [lax.shift_right_logical(v, 4), v & (L - 1)], one)

    plsc.subcore_barrier()
    for j in range(5):
        pltpu.sync_copy(hist_v.at[pl.ds(j * CH, CH)],
                        hd_s.at[idxio.at[j]], add=True)
    plsc.subcore_barrier()

    pltpu.sync_copy(hs_s.at[pl.ds(hb, HRS)], deg_v)
    for j in range(HRS):
        nsrc_v[j] = _rsqrt16(jnp.maximum(deg_v[j], 1.0))
    pltpu.sync_copy(hd_s.at[pl.ds(hb, HRS)], deg_v)
    for j in range(HRS):
        deg_v[j] = _rsqrt16(jnp.maximum(deg_v[j], 1.0))

    @pl.when(cid == 0)
    def _wn():
        pltpu.sync_copy(deg_v, nrm_hbm.at[pl.ds(hb, HRS)])

    def _half(x_hbm, xs_hbm):
        for p in range(5):
            buf = rows[p % 2]
            pltpu.sync_copy(x_hbm.at[pl.ds(rb + p * CH, CH)], buf)

            @pl.loop(0, CH)
            def _scale(r):
                g = p * CH + r
                nv = plsc.load_gather(
                    nsrc_v, [jnp.full((L,), lax.shift_right_logical(g, 4),
                                      jnp.int32),
                             jnp.full((L,), g & (L - 1), jnp.int32)])
                for k in range(DH // L):
                    buf[r, pl.ds(k * L, L)] = buf[r, pl.ds(k * L, L)] * nv

            pltpu.sync_copy(buf, xs_hbm.at[pl.ds(rb + p * CH, CH)])

        plsc.subcore_barrier()

        for j in range(NB):
            pltpu.async_copy(xs_hbm.at[src_v.at[j]], rows[j], sems[j])

        @pl.loop(0, NCH // NB)
        def _chunks(i):
            cc = i * NB
            for j in range(NB):
                pltpu.make_async_copy(
                    xs_hbm.at[src_v.at[cc + j]], rows[j], sems[j]).wait()
                pltpu.async_copy(
                    rows[j], acc.at[dst_v.at[cc + j]], sems[j], add=True)
            for j in range(NB):
                pltpu.make_async_copy(
                    rows[j], acc.at[dst_v.at[cc + j]], sems[j]).wait()

                @pl.when(cc + NB + j < NCH)
                def _():
                    pltpu.async_copy(
                        xs_hbm.at[src_v.at[cc + NB + j]], rows[j], sems[j])

    @pl.when(cid == 0)
    def _lo():
        _half(x_lo_hbm, xs_lo_hbm)

    @pl.when(cid == 1)
    def _hi():
        _half(x_hi_hbm, xs_hi_hbm)

    plsc.subcore_barrier()
    pltpu.sync_copy(acc.at[pl.ds(rb, RPS)], out_hbm.at[cid, pl.ds(rb, RPS)])


def _final_body(p_ref, nrm_ref, w_ref, b_ref, o_ref):
    s = jnp.concatenate([p_ref[0], p_ref[1]], axis=-1)
    h = jnp.dot(s, w_ref[...], preferred_element_type=jnp.float32,
                precision=lax.Precision.HIGHEST)
    o_ref[...] = h * nrm_ref[0][:, None] + b_ref[0][None, :]


_BLKO = 1024
_tc_final = pl.pallas_call(
    _final_body,
    grid=(pl.cdiv(N_NODES, _BLKO),),
    in_specs=[
        pl.BlockSpec((NC, _BLKO, DH), lambda i: (0, i, 0)),
        pl.BlockSpec((1, _BLKO), lambda i: (0, i)),
        pl.BlockSpec((D, D), lambda i: (0, 0)),
        pl.BlockSpec((1, D), lambda i: (0, 0)),
    ],
    out_specs=pl.BlockSpec((_BLKO, D), lambda i: (i, 0)),
    out_shape=jax.ShapeDtypeStruct((N_NODES, D), jnp.float32),
)


def kernel(x, edge_index, W, b):
    src = edge_index[0]
    dst = edge_index[1]
    pad = jnp.full((EP - N_EDGES,), N_NODES, dtype=jnp.int32)
    src_p = jnp.concatenate([src, pad]).reshape(NS, NCH, CH)
    dst_p = jnp.concatenate([dst, pad]).reshape(NS, NCH, CH)

    x_p = jnp.concatenate([x, jnp.zeros((NP - N_NODES, D), x.dtype)])
    z1 = jnp.zeros((NPR, L), jnp.float32)
    z2 = jnp.zeros((RPS, DH), jnp.float32)

    parts, _, _, nrm = _sc_fused(x_p[:, :DH], x_p[:, DH:], src_p, dst_p,
                                 z1, z2)

    return _tc_final(parts, nrm.reshape(1, NP), W, b.reshape(1, D))


# speedup vs baseline: 11.6808x; 1.0022x over previous
import functools

import jax
import jax.numpy as jnp
from jax import lax
from jax.experimental import pallas as pl
from jax.experimental.pallas import tpu as pltpu
from jax.experimental.pallas import tpu_sc as plsc

NC = 2
NS = 16
L = 16

N_NODES = 10000
D = 128
DH = D // NC
NP = 10240
NPR = NP // L
N_EDGES = 320000
CH = 128
NCH = 160
EPT = CH * NCH
EP = NS * EPT
RPS = NP // NS
HRS = NPR // NS
NB = 4

_mesh = plsc.VectorSubcoreMesh(core_axis_name="c", subcore_axis_name="s")


def _rsqrt16(d):
    i = plsc.bitcast(d, jnp.int32)
    i = 0x5F3759DF - lax.shift_right_logical(i, 1)
    y = plsc.bitcast(i, jnp.float32)
    for _ in range(3):
        y = y * (1.5 - 0.5 * d * y * y)
    return y


@functools.partial(
    pl.kernel,
    out_type=[
        jax.ShapeDtypeStruct((NC, NP, DH), jnp.float32),
        jax.ShapeDtypeStruct((NP, DH), jnp.float32),
        jax.ShapeDtypeStruct((NP, DH), jnp.float32),
        jax.ShapeDtypeStruct((NPR, L), jnp.float32),
    ],
    mesh=_mesh,
    scratch_types=[
        pltpu.VMEM((NCH, CH), jnp.int32),
        pltpu.VMEM((NCH, CH), jnp.int32),
        pltpu.VMEM((CH, DH), jnp.float32),
        pltpu.VMEM((CH, DH), jnp.float32),
        pltpu.VMEM((CH, DH), jnp.float32),
        pltpu.VMEM((CH, DH), jnp.float32),
        pltpu.VMEM((NPR, L), jnp.float32),
        pltpu.VMEM((HRS, L), jnp.float32),
        pltpu.VMEM((HRS, L), jnp.float32),
        pltpu.VMEM((5, CH), jnp.int32),
        pltpu.VMEM_SHARED((NP, DH), jnp.float32),
        pltpu.VMEM_SHARED((NPR, L), jnp.float32),
        pltpu.VMEM_SHARED((NPR, L), jnp.float32),
        pltpu.SemaphoreType.DMA,
        pltpu.SemaphoreType.DMA,
        pltpu.SemaphoreType.DMA,
        pltpu.SemaphoreType.DMA,
    ],
    compiler_params=pltpu.CompilerParams(needs_layout_passes=False,
                                         use_tc_tiling_on_sc=False),
)
def _sc_fused(x_lo_hbm, x_hi_hbm, src_hbm, dst_hbm, z1_hbm, z2_hbm,
              out_hbm, xs_lo_hbm, xs_hi_hbm, nrm_hbm,
              src_v, dst_v, rows0, rows1, rows2, rows3,
              hist_v, deg_v, nsrc_v, idxio,
              acc, hs_s, hd_s, sem0, sem1, sem2, sem3):
    cid = lax.axis_index("c")
    sid = lax.axis_index("s")
    rb = sid * RPS
    hb = sid * HRS
    rows = (rows0, rows1, rows2, rows3)
    sems = (sem0, sem1, sem2, sem3)
    one = jnp.ones((L,), jnp.float32)

    c0 = pltpu.async_copy(z2_hbm, acc.at[pl.ds(rb, RPS)], sem0)
    c1 = pltpu.async_copy(src_hbm.at[sid], src_v, sem1)
    c2 = pltpu.async_copy(dst_hbm.at[sid], dst_v, sem2)
    c3 = pltpu.async_copy(z1_hbm, hist_v, sem3)
    c4 = pltpu.async_copy(z1_hbm.at[pl.ds(hb, HRS)],
                          hs_s.at[pl.ds(hb, HRS)], sem0)
    c5 = pltpu.async_copy(z1_hbm.at[pl.ds(hb, HRS)],
                          hd_s.at[pl.ds(hb, HRS)], sem1)
    for j in range(5):
        for k in range(CH // L):
            idxio[j, pl.ds(k * L, L)] = (
                lax.iota(jnp.int32, L) + (j * CH + k * L))
    c0.wait()
    c1.wait()
    c2.wait()
    c3.wait()
    c4.wait()
    c5.wait()

    @pl.loop(0, EPT // L, unroll=4)
    def _hist_src(i):
        o = pl.multiple_of(i * L, L)
        c = o // CH
        k = o - c * CH
        v = src_v[c, pl.ds(k, L)]
        plsc.addupdate_scatter(
            hist_v, [lax.shift_right_logical(v, 4), v & (L - 1)], one)

    plsc.subcore_barrier()
    for j in range(5):
        pltpu.sync_copy(hist_v.at[pl.ds(j * CH, CH)],
                        hs_s.at[idxio.at[j]], add=True)
    pltpu.sync_copy(z1_hbm, hist_v)

    @pl.loop(0, EPT // L, unroll=4)
    def _hist_dst(i):
        o = pl.multiple_of(i * L, L)
        c = o // CH
        k = o - c * CH
        v = dst_v[c, pl.ds(k, L)]
        plsc.addupdate_scatter(
            hist_v, [lax.shift_right_logical(v, 4), v & (L - 1)], one)

    plsc.subcore_barrier()
    for j in range(5):
        pltpu.sync_copy(hist_v.at[pl.ds(j * CH, CH)],
                        hd_s.at[idxio.at[j]], add=True)
    plsc.subcore_barrier()

    pltpu.sync_copy(hs_s.at[pl.ds(hb, HRS)], deg_v)
    for j in range(HRS):
        nsrc_v[j] = _rsqrt16(jnp.maximum(deg_v[j], 1.0))
    pltpu.sync_copy(hd_s.at[pl.ds(hb, HRS)], deg_v)
    for j in range(HRS):
        deg_v[j] = _rsqrt16(jnp.maximum(deg_v[j], 1.0))

    @pl.when(cid == 1)
    def _wn():
        pltpu.sync_copy(deg_v, nrm_hbm.at[pl.ds(hb, HRS)])

    def _half(x_hbm, xs_hbm):
        for p in range(5):
            buf = rows[p % 2]
            pltpu.sync_copy(x_hbm.at[pl.ds(rb + p * CH, CH)], buf)

            @pl.loop(0, CH)
            def _scale(r):
                g = p * CH + r
                nv = plsc.load_gather(
                    nsrc_v, [jnp.full((L,), lax.shift_right_logical(g, 4),
                                      jnp.int32),
                             jnp.full((L,), g & (L - 1), jnp.int32)])
                for k in range(DH // L):
                    buf[r, pl.ds(k * L, L)] = buf[r, pl.ds(k * L, L)] * nv

            pltpu.sync_copy(buf, xs_hbm.at[pl.ds(rb + p * CH, CH)])

        plsc.subcore_barrier()

        for j in range(NB):
            pltpu.async_copy(xs_hbm.at[src_v.at[j]], rows[j], sems[j])

        @pl.loop(0, NCH // NB)
        def _chunks(i):
            cc = i * NB
            for j in range(NB):
                pltpu.make_async_copy(
                    xs_hbm.at[src_v.at[cc + j]], rows[j], sems[j]).wait()
                pltpu.async_copy(
                    rows[j], acc.at[dst_v.at[cc + j]], sems[j], add=True)
            for j in range(NB):
                pltpu.make_async_copy(
                    rows[j], acc.at[dst_v.at[cc + j]], sems[j]).wait()

                @pl.when(cc + NB + j < NCH)
                def _():
                    pltpu.async_copy(
                        xs_hbm.at[src_v.at[cc + NB + j]], rows[j], sems[j])

    @pl.when(cid == 0)
    def _lo():
        _half(x_lo_hbm, xs_lo_hbm)

    @pl.when(cid == 1)
    def _hi():
        _half(x_hi_hbm, xs_hi_hbm)

    plsc.subcore_barrier()
    pltpu.sync_copy(acc.at[pl.ds(rb, RPS)], out_hbm.at[cid, pl.ds(rb, RPS)])


def _final_body(p_ref, nrm_ref, w_ref, b_ref, o_ref):
    s = jnp.concatenate([p_ref[0], p_ref[1]], axis=-1)
    h = jnp.dot(s, w_ref[...], preferred_element_type=jnp.float32,
                precision=lax.Precision.HIGHEST)
    o_ref[...] = h * nrm_ref[0][:, None] + b_ref[0][None, :]


_BLKO = 1024
_tc_final = pl.pallas_call(
    _final_body,
    grid=(pl.cdiv(N_NODES, _BLKO),),
    in_specs=[
        pl.BlockSpec((NC, _BLKO, DH), lambda i: (0, i, 0)),
        pl.BlockSpec((1, _BLKO), lambda i: (0, i)),
        pl.BlockSpec((D, D), lambda i: (0, 0)),
        pl.BlockSpec((1, D), lambda i: (0, 0)),
    ],
    out_specs=pl.BlockSpec((_BLKO, D), lambda i: (i, 0)),
    out_shape=jax.ShapeDtypeStruct((N_NODES, D), jnp.float32),
)


def kernel(x, edge_index, W, b):
    src = edge_index[0]
    dst = edge_index[1]
    pad = jnp.full((EP - N_EDGES,), N_NODES, dtype=jnp.int32)
    src_p = jnp.concatenate([src, pad]).reshape(NS, NCH, CH)
    dst_p = jnp.concatenate([dst, pad]).reshape(NS, NCH, CH)

    x_p = jnp.concatenate([x, jnp.zeros((NP - N_NODES, D), x.dtype)])
    z1 = jnp.zeros((NPR, L), jnp.float32)
    z2 = jnp.zeros((RPS, DH), jnp.float32)

    parts, _, _, nrm = _sc_fused(x_p[:, :DH], x_p[:, DH:], src_p, dst_p,
                                 z1, z2)

    return _tc_final(parts, nrm.reshape(1, NP), W, b.reshape(1, D))


# speedup vs baseline: 11.8696x; 1.0162x over previous
import functools

import jax
import jax.numpy as jnp
from jax import lax
from jax.experimental import pallas as pl
from jax.experimental.pallas import tpu as pltpu
from jax.experimental.pallas import tpu_sc as plsc

NC = 2
NS = 16
L = 16

N_NODES = 10000
D = 128
DH = D // NC
NP = 10240
NPR = NP // L
N_EDGES = 320000
CH = 128
NCH = 160
EPT = CH * NCH
EP = NS * EPT
RPS = NP // NS
HRS = NPR // NS
NB = 4

_mesh = plsc.VectorSubcoreMesh(core_axis_name="c", subcore_axis_name="s")


def _rsqrt16(d):
    i = plsc.bitcast(d, jnp.int32)
    i = 0x5F3759DF - lax.shift_right_logical(i, 1)
    y = plsc.bitcast(i, jnp.float32)
    for _ in range(3):
        y = y * (1.5 - 0.5 * d * y * y)
    return y


@functools.partial(
    pl.kernel,
    out_type=[
        jax.ShapeDtypeStruct((NC, NP, DH), jnp.float32),
        jax.ShapeDtypeStruct((NP, DH), jnp.float32),
        jax.ShapeDtypeStruct((NP, DH), jnp.float32),
        jax.ShapeDtypeStruct((NC, NPR, L), jnp.float32),
    ],
    mesh=_mesh,
    scratch_types=[
        pltpu.VMEM((NCH, CH), jnp.int32),
        pltpu.VMEM((NCH, CH), jnp.int32),
        pltpu.VMEM((CH, DH), jnp.float32),
        pltpu.VMEM((CH, DH), jnp.float32),
        pltpu.VMEM((CH, DH), jnp.float32),
        pltpu.VMEM((CH, DH), jnp.float32),
        pltpu.VMEM((NPR, L), jnp.float32),
        pltpu.VMEM((HRS, L), jnp.float32),
        pltpu.VMEM((HRS, L), jnp.float32),
        pltpu.VMEM((5, CH), jnp.int32),
        pltpu.VMEM_SHARED((NP, DH), jnp.float32),
        pltpu.VMEM_SHARED((NPR, L), jnp.float32),
        pltpu.VMEM_SHARED((NPR, L), jnp.float32),
        pltpu.SemaphoreType.DMA,
        pltpu.SemaphoreType.DMA,
        pltpu.SemaphoreType.DMA,
        pltpu.SemaphoreType.DMA,
    ],
    compiler_params=pltpu.CompilerParams(needs_layout_passes=False,
                                         use_tc_tiling_on_sc=False),
)
def _sc_fused(x_lo_hbm, x_hi_hbm, src_hbm, dst_hbm, z1_hbm, z2_hbm,
              out_hbm, xs_lo_hbm, xs_hi_hbm, hd_hbm,
              src_v, dst_v, rows0, rows1, rows2, rows3,
              hist_v, deg_v, nsrc_v, idxio,
              acc, hs_s, hd_s, sem0, sem1, sem2, sem3):
    cid = lax.axis_index("c")
    sid = lax.axis_index("s")
    rb = sid * RPS
    hb = sid * HRS
    rows = (rows0, rows1, rows2, rows3)
    sems = (sem0, sem1, sem2, sem3)
    one = jnp.ones((L,), jnp.float32)

    c0 = pltpu.async_copy(z2_hbm, acc.at[pl.ds(rb, RPS)], sem0)
    c1 = pltpu.async_copy(src_hbm.at[sid], src_v, sem1)
    c2 = pltpu.async_copy(dst_hbm.at[sid], dst_v, sem2)
    c3 = pltpu.async_copy(z1_hbm, hist_v, sem3)
    c4 = pltpu.async_copy(z1_hbm.at[pl.ds(hb, HRS)],
                          hs_s.at[pl.ds(hb, HRS)], sem0)
    c5 = pltpu.async_copy(z1_hbm.at[pl.ds(hb, HRS)],
                          hd_s.at[pl.ds(hb, HRS)], sem1)
    for j in range(5):
        for k in range(CH // L):
            idxio[j, pl.ds(k * L, L)] = (
                lax.iota(jnp.int32, L) + (j * CH + k * L))
    c0.wait()
    c1.wait()
    c2.wait()
    c3.wait()
    c4.wait()
    c5.wait()

    @pl.loop(0, EPT // L, unroll=8)
    def _hist_src(i):
        o = pl.multiple_of(i * L, L)
        c = o // CH
        k = o - c * CH
        v = src_v[c, pl.ds(k, L)]
        plsc.addupdate_scatter(
            hist_v, [lax.shift_right_logical(v, 4), v & (L - 1)], one)

    plsc.subcore_barrier()
    for j in range(5):
        pltpu.sync_copy(hist_v.at[pl.ds(j * CH, CH)],
                        hs_s.at[idxio.at[j]], add=True)
    pltpu.sync_copy(z1_hbm, hist_v)

    @pl.loop(0, EPT // L // 2, unroll=8)
    def _hist_dst(i):
        o = pl.multiple_of(i * L, L) + cid * (EPT // 2)
        c = o // CH
        k = o - c * CH
        v = dst_v[c, pl.ds(k, L)]
        plsc.addupdate_scatter(
            hist_v, [lax.shift_right_logical(v, 4), v & (L - 1)], one)

    plsc.subcore_barrier()
    for j in range(5):
        pltpu.sync_copy(hist_v.at[pl.ds(j * CH, CH)],
                        hd_s.at[idxio.at[j]], add=True)
    plsc.subcore_barrier()

    pltpu.sync_copy(hs_s.at[pl.ds(hb, HRS)], deg_v)
    for j in range(HRS):
        nsrc_v[j] = _rsqrt16(jnp.maximum(deg_v[j], 1.0))
    pltpu.sync_copy(hd_s.at[pl.ds(hb, HRS)], hd_hbm.at[cid, pl.ds(hb, HRS)])

    def _half(x_hbm, xs_hbm):
        pltpu.async_copy(x_hbm.at[pl.ds(rb, CH)], rows[0], sems[0])
        for p in range(5):
            b = p % 2
            buf = rows[b]
            pltpu.make_async_copy(
                x_hbm.at[pl.ds(rb + p * CH, CH)], buf, sems[b]).wait()
            if p >= 1:
                pltpu.make_async_copy(
                    rows[1 - b], xs_hbm.at[pl.ds(rb + (p - 1) * CH, CH)],
                    sems[2 + (1 - b)]).wait()
            if p + 1 < 5:
                pltpu.async_copy(x_hbm.at[pl.ds(rb + (p + 1) * CH, CH)],
                                 rows[1 - b], sems[1 - b])

            @pl.loop(0, CH)
            def _scale(r):
                g = p * CH + r
                nv = plsc.load_gather(
                    nsrc_v, [jnp.full((L,), lax.shift_right_logical(g, 4),
                                      jnp.int32),
                             jnp.full((L,), g & (L - 1), jnp.int32)])
                for k in range(DH // L):
                    buf[r, pl.ds(k * L, L)] = buf[r, pl.ds(k * L, L)] * nv

            pltpu.async_copy(buf, xs_hbm.at[pl.ds(rb + p * CH, CH)],
                             sems[2 + b])
        pltpu.make_async_copy(
            rows[0], xs_hbm.at[pl.ds(rb + 4 * CH, CH)], sems[2]).wait()

        plsc.subcore_barrier()

        for j in range(NB):
            pltpu.async_copy(xs_hbm.at[src_v.at[j]], rows[j], sems[j])

        @pl.loop(0, NCH // NB)
        def _chunks(i):
            cc = i * NB
            for j in range(NB):
                pltpu.make_async_copy(
                    xs_hbm.at[src_v.at[cc + j]], rows[j], sems[j]).wait()
                pltpu.async_copy(
                    rows[j], acc.at[dst_v.at[cc + j]], sems[j], add=True)
            for j in range(NB):
                pltpu.make_async_copy(
                    rows[j], acc.at[dst_v.at[cc + j]], sems[j]).wait()

                @pl.when(cc + NB + j < NCH)
                def _():
                    pltpu.async_copy(
                        xs_hbm.at[src_v.at[cc + NB + j]], rows[j], sems[j])

    @pl.when(cid == 0)
    def _lo():
        _half(x_lo_hbm, xs_lo_hbm)

    @pl.when(cid == 1)
    def _hi():
        _half(x_hi_hbm, xs_hi_hbm)

    plsc.subcore_barrier()
    pltpu.sync_copy(acc.at[pl.ds(rb, RPS)], out_hbm.at[cid, pl.ds(rb, RPS)])


def _final_body(p_ref, hd_ref, w_ref, b_ref, o_ref):
    s = jnp.concatenate([p_ref[0], p_ref[1]], axis=-1)
    d = hd_ref[0] + hd_ref[1]
    nrm = lax.rsqrt(jnp.maximum(d, 1.0))
    h = jnp.dot(s, w_ref[...], preferred_element_type=jnp.float32,
                precision=lax.Precision.HIGHEST)
    o_ref[...] = h * nrm[:, None] + b_ref[0][None, :]


_BLKO = 1024
_tc_final = pl.pallas_call(
    _final_body,
    grid=(pl.cdiv(N_NODES, _BLKO),),
    in_specs=[
        pl.BlockSpec((NC, _BLKO, DH), lambda i: (0, i, 0)),
        pl.BlockSpec((NC, _BLKO), lambda i: (0, i)),
        pl.BlockSpec((D, D), lambda i: (0, 0)),
        pl.BlockSpec((1, D), lambda i: (0, 0)),
    ],
    out_specs=pl.BlockSpec((_BLKO, D), lambda i: (i, 0)),
    out_shape=jax.ShapeDtypeStruct((N_NODES, D), jnp.float32),
)


def kernel(x, edge_index, W, b):
    src = edge_index[0]
    dst = edge_index[1]
    pad = jnp.full((EP - N_EDGES,), N_NODES, dtype=jnp.int32)
    src_p = jnp.concatenate([src, pad]).reshape(NS, NCH, CH)
    dst_p = jnp.concatenate([dst, pad]).reshape(NS, NCH, CH)

    x_p = jnp.concatenate([x, jnp.zeros((NP - N_NODES, D), x.dtype)])
    z1 = jnp.zeros((NPR, L), jnp.float32)
    z2 = jnp.zeros((RPS, DH), jnp.float32)

    parts, _, _, hd = _sc_fused(x_p[:, :DH], x_p[:, DH:], src_p, dst_p,
                                z1, z2)

    return _tc_final(parts, hd.reshape(NC, NP), W, b.reshape(1, D))
